# baseline (speedup 1.0000x reference)
import functools

import jax
import jax.numpy as jnp
from jax import lax
from jax.experimental import pallas as pl
from jax.experimental.pallas import tpu as pltpu
from jax.experimental.pallas import tpu_sc as plsc

_NUM_BINS = 100000
_ROWS, _COLS = 16384, 100
_TOTAL = _ROWS * _COLS
_NC, _NS, _L = 2, 16, 16
_NW = _NC * _NS
_PER_W = _TOTAL // _NW
_NVEC = _PER_W // _L

def _hash16(x):
    c = jnp.uint32(0x45D9F3B)
    x = (x ^ (x >> 16)) * c
    x = (x ^ (x >> 16)) * c
    x = x ^ (x >> 16)
    return (x % jnp.uint32(_NUM_BINS)).astype(jnp.int32)


@functools.partial(
    pl.kernel,
    out_type=jax.ShapeDtypeStruct((_TOTAL,), jnp.int32),
    mesh=plsc.VectorSubcoreMesh(core_axis_name="c", subcore_axis_name="s"),
    scratch_types=[
        pltpu.VMEM((_PER_W,), jnp.int32),
        pltpu.VMEM((_PER_W,), jnp.int32),
    ],
)
def _hash_sc(x_hbm, out_hbm, in_v, out_v):
    wid = lax.axis_index("s") * _NC + lax.axis_index("c")
    base = wid * _PER_W
    pltpu.sync_copy(x_hbm.at[pl.ds(base, _PER_W)], in_v)

    @pl.loop(0, _NVEC, unroll=8)
    def _(i):
        x = in_v[pl.ds(i * _L, _L)].astype(jnp.uint32)
        out_v[pl.ds(i * _L, _L)] = _hash16(x)

    pltpu.sync_copy(out_v, out_hbm.at[pl.ds(base, _PER_W)])


def kernel(inputs):
    flat = inputs.reshape(_TOTAL)
    return _hash_sc(flat).reshape(_ROWS, _COLS)
